# baseline (speedup 1.0000x reference)
import jax
import jax.numpy as jnp
from jax import lax
from jax.experimental import pallas as pl
from jax.experimental.pallas import tpu as pltpu
from jax.experimental.pallas import tpu_sc as plsc

HIDDEN = 64
LANES = 16
NC, NS = 2, 16
NW = NC * NS
NB = 4
C0 = 128
SCALE = 8.0


def _build(B, S):
    assert B % NW == 0
    rpw = B // NW
    assert rpw % NB == 0
    steps = rpw // NB
    c1 = S - C0
    mesh = plsc.VectorSubcoreMesh(
        core_axis_name="c", subcore_axis_name="s",
        num_cores=NC, num_subcores=NS)

    def body(x_hbm, table_hbm, out_hbm, idx_v, gbuf, sbuf, gsem, ssem):
        wid = lax.axis_index("s") * NC + lax.axis_index("c")
        rbase = wid * rpw
        pltpu.sync_copy(x_hbm.at[pl.ds(rbase, rpw)], idx_v)

        def fire_gathers(slot, r):
            pltpu.async_copy(
                table_hbm.at[idx_v.at[r, pl.ds(0, C0)]],
                gbuf.at[slot, pl.ds(0, C0)], gsem.at[slot])
            pltpu.async_copy(
                table_hbm.at[idx_v.at[r, pl.ds(C0, c1)]],
                gbuf.at[slot, pl.ds(C0, c1)], gsem.at[slot])

        def wait_gathers(slot, r):
            pltpu.make_async_copy(
                table_hbm.at[idx_v.at[r, pl.ds(0, C0)]],
                gbuf.at[slot, pl.ds(0, C0)], gsem.at[slot]).wait()
            pltpu.make_async_copy(
                table_hbm.at[idx_v.at[r, pl.ds(C0, c1)]],
                gbuf.at[slot, pl.ds(C0, c1)], gsem.at[slot]).wait()

        for b in range(NB):
            fire_gathers(b, b)

        @pl.loop(0, steps)
        def _step(step):
            for b in range(NB):
                r = step * NB + b
                wait_gathers(b, r)

                @pl.when(step > 0)
                def _():
                    pltpu.make_async_copy(
                        sbuf.at[b], out_hbm.at[rbase], ssem.at[b]).wait()

                @pl.loop(0, S)
                def _row(t):
                    for j in range(HIDDEN // LANES):
                        sl = pl.ds(j * LANES, LANES)
                        sbuf[b, t, sl] = gbuf[b, t, sl] * SCALE

                @pl.when(step < steps - 1)
                def _():
                    fire_gathers(b, r + NB)

                pltpu.async_copy(sbuf.at[b], out_hbm.at[rbase + r],
                                 ssem.at[b])

        for b in range(NB):
            pltpu.make_async_copy(
                sbuf.at[b], out_hbm.at[rbase], ssem.at[b]).wait()

    return pl.kernel(
        body,
        out_type=jax.ShapeDtypeStruct((B, S, HIDDEN), jnp.float32),
        mesh=mesh,
        scratch_types=[
            pltpu.VMEM((rpw, S), jnp.int32),
            pltpu.VMEM((NB, S, HIDDEN), jnp.float32),
            pltpu.VMEM((NB, S, HIDDEN), jnp.float32),
            pltpu.SemaphoreType.DMA((NB,)),
            pltpu.SemaphoreType.DMA((NB,)),
        ],
        compiler_params=pltpu.CompilerParams(use_tc_tiling_on_sc=False),
    )


def kernel(x, table):
    b, s = x.shape
    return _build(b, s)(x.astype(jnp.int32), table)


# speedup vs baseline: 1.1098x; 1.1098x over previous
import jax
import jax.numpy as jnp
from jax import lax
from jax.experimental import pallas as pl
from jax.experimental.pallas import tpu as pltpu
from jax.experimental.pallas import tpu_sc as plsc

HIDDEN = 64
WIDE = 128
LANES = 16
NC, NS = 2, 16
NW = NC * NS
NB = 2
NCHUNK = 2
C0 = 128
SCALE = 8.0


def _build(B, S):
    assert B % (NW * NCHUNK) == 0
    rpw = B // NW
    rpc = rpw // NCHUNK
    assert rpc % NB == 0
    steps = rpc // NB
    c1 = S - C0
    mesh = plsc.VectorSubcoreMesh(
        core_axis_name="c", subcore_axis_name="s",
        num_cores=NC, num_subcores=NS)

    def body(x_hbm, table_hbm, out_hbm, idx_v, gbuf, sbuf, gsem, ssem):
        wid = lax.axis_index("s") * NC + lax.axis_index("c")
        rbase = wid * rpw

        def fire_gathers(slot, r):
            pltpu.async_copy(
                table_hbm.at[idx_v.at[pl.ds(r * S, C0)]],
                gbuf.at[slot, pl.ds(0, C0)], gsem.at[slot])
            pltpu.async_copy(
                table_hbm.at[idx_v.at[pl.ds(r * S + C0, c1)]],
                gbuf.at[slot, pl.ds(C0, c1)], gsem.at[slot])

        def wait_gathers(slot, r):
            pltpu.make_async_copy(
                table_hbm.at[idx_v.at[pl.ds(r * S, C0)]],
                gbuf.at[slot, pl.ds(0, C0)], gsem.at[slot]).wait()
            pltpu.make_async_copy(
                table_hbm.at[idx_v.at[pl.ds(r * S + C0, c1)]],
                gbuf.at[slot, pl.ds(C0, c1)], gsem.at[slot]).wait()

        for chunk in range(NCHUNK):
            cbase = rbase + chunk * rpc
            pltpu.sync_copy(
                x_hbm.at[pl.ds(cbase * S, rpc * S)], idx_v)

            for b in range(NB):
                fire_gathers(b, b)

            @pl.loop(0, steps)
            def _step(step):
                for b in range(NB):
                    r = step * NB + b
                    wait_gathers(b, r)

                    @pl.when(jnp.logical_or(step > 0, chunk > 0))
                    def _():
                        pltpu.make_async_copy(
                            sbuf.at[b], out_hbm.at[rbase], ssem.at[b]).wait()

                    @pl.loop(0, S)
                    def _row(t):
                        for j in range(HIDDEN // LANES):
                            sl = pl.ds(j * LANES, LANES)
                            sbuf[b, t, sl] = gbuf[b, t, sl] * SCALE

                    pltpu.async_copy(
                        sbuf.at[b], out_hbm.at[cbase + r], ssem.at[b])

                    @pl.when(step < steps - 1)
                    def _():
                        fire_gathers(b, r + NB)

        for b in range(NB):
            pltpu.make_async_copy(
                sbuf.at[b], out_hbm.at[rbase], ssem.at[b]).wait()

    return pl.kernel(
        body,
        out_type=jax.ShapeDtypeStruct((B, S, HIDDEN), jnp.float32),
        mesh=mesh,
        scratch_types=[
            pltpu.VMEM((B // NW // NCHUNK * S,), jnp.int32),
            pltpu.VMEM((NB, S, WIDE), jnp.float32),
            pltpu.VMEM((NB, S, HIDDEN), jnp.float32),
            pltpu.SemaphoreType.DMA((NB,)),
            pltpu.SemaphoreType.DMA((NB,)),
        ],
        compiler_params=pltpu.CompilerParams(use_tc_tiling_on_sc=True),
    )


def kernel(x, table):
    b, s = x.shape
    v, h = table.shape
    table_p = jnp.pad(table, ((0, 0), (0, WIDE - h)))
    x_flat = x.astype(jnp.int32).reshape(b * s)
    return _build(b, s)(x_flat, table_p)
